# baseline (speedup 1.0000x reference)
import functools

import jax
import jax.numpy as jnp
from jax import lax
from jax.experimental import pallas as pl
from jax.experimental.pallas import tpu as pltpu
from jax.experimental.pallas import tpu_sc as plsc

B = 16384
D = 32
L = 16
NC = 2
NS = 16
NW = NC * NS
BPW = B // NW
NB = 62500
ROWS = 16 * NB
NG = D // 2
CHUNK = 128
NCHUNK = BPW // CHUNK
REG = BPW


def _granule_view(table):
    tb = table.astype(jnp.bfloat16)
    be = jax.lax.bitcast_convert_type(tb[:, 0::2], jnp.uint16)
    bo = jax.lax.bitcast_convert_type(tb[:, 1::2], jnp.uint16)
    w = jnp.bitwise_or(be.astype(jnp.uint32),
                       jnp.left_shift(bo.astype(jnp.uint32), 16))
    x = w.T.reshape(ROWS, 16)
    return jax.lax.bitcast_convert_type(x, jnp.int32)


def _prep_indices(raw_i, base_i, word_i, nt):
    for t in range(nt):
        for m in range(BPW // L):
            sl = pl.ds(m * L, L)
            n = raw_i[t, sl]
            base_i[t, sl] = lax.shift_right_logical(n, 4)
            word_i[t, sl] = jnp.bitwise_and(n, 15)


def _stage_and_fire(tables, g, par, sem, base_i, gidx, cols, nt):
    offs = g * NB
    for t in range(nt):
        for j in range(NCHUNK):
            row = (par * nt + t) * NCHUNK + j
            for m in range(CHUNK // L):
                sl = pl.ds(j * CHUNK + m * L, L)
                gidx[row, pl.ds(m * L, L)] = base_i[t, sl] + offs
    for t in range(nt):
        for j in range(NCHUNK):
            row = (par * nt + t) * NCHUNK + j
            dst = pl.ds((par * nt + t) * REG + j * CHUNK, CHUNK)
            pltpu.async_copy(tables[t].at[gidx.at[row]], cols.at[dst], sem)


def _drain(x_any, par, sem, cols, nt):
    for t in range(nt):
        reg = pl.ds((par * nt + t) * REG, REG)
        pltpu.make_async_copy(x_any.at[pl.ds(0, REG)], cols.at[reg],
                              sem).wait()


def _pair(cols, ridx, word):
    w = plsc.load_gather(cols, [ridx, word])
    lo = plsc.bitcast(lax.shift_left(w, 16), jnp.float32)
    hi = plsc.bitcast(jnp.bitwise_and(w, jnp.int32(-65536)), jnp.float32)
    return lo, hi


def _k1_body(subj_hbm, obj_hbm, x_e, p_hbm,
             raw_i, base_i, word_i, gidx, cols, prod_v, sem0, sem1):
    wid = lax.axis_index("s") * NC + lax.axis_index("c")
    pltpu.sync_copy(subj_hbm.at[wid], raw_i.at[0])
    pltpu.sync_copy(obj_hbm.at[wid], raw_i.at[1])
    _prep_indices(raw_i, base_i, word_i, 2)
    tables = (x_e, x_e)

    def accumulate(g, par):
        for m in range(BPW // L):
            sl = pl.ds(m * L, L)
            rloc = lax.iota(jnp.int32, L) + m * L
            s_lo, s_hi = _pair(cols, rloc + (par * 2 + 0) * REG,
                               word_i[0, sl])
            o_lo, o_hi = _pair(cols, rloc + (par * 2 + 1) * REG,
                               word_i[1, sl])
            prod_v[par, 0, sl] = s_lo * o_lo
            prod_v[par, 1, sl] = s_hi * o_hi
        pltpu.sync_copy(prod_v.at[par], p_hbm.at[wid, pl.ds(g * 2, 2)])

    _stage_and_fire(tables, 0, 0, sem0, base_i, gidx, cols, 2)
    _stage_and_fire(tables, 1, 1, sem1, base_i, gidx, cols, 2)

    def pipe_body(k, _):
        g_e = k * 2
        _drain(x_e, 0, sem0, cols, 2)
        accumulate(g_e, 0)

        @pl.when(k < NG // 2 - 1)
        def _fe():
            _stage_and_fire(tables, g_e + 2, 0, sem0, base_i, gidx, cols, 2)

        _drain(x_e, 1, sem1, cols, 2)
        accumulate(g_e + 1, 1)

        @pl.when(k < NG // 2 - 1)
        def _fo():
            _stage_and_fire(tables, g_e + 3, 1, sem1, base_i, gidx, cols, 2)

        return 0

    lax.fori_loop(0, NG // 2, pipe_body, 0)


def _k2_body(rel_hbm, x_r, p_hbm, out_hbm,
             raw_i, base_i, word_i, gidx, cols, prod_v, out_v, sem0, sem1):
    wid = lax.axis_index("s") * NC + lax.axis_index("c")
    pltpu.sync_copy(rel_hbm.at[wid], raw_i.at[0])
    pltpu.sync_copy(p_hbm.at[wid], prod_v)
    _prep_indices(raw_i, base_i, word_i, 1)
    tables = (x_r,)

    zero = jnp.zeros((L,), jnp.float32)
    for m in range(BPW // L):
        out_v[pl.ds(m * L, L)] = zero

    def accumulate(g, par):
        d0 = g * 2
        for m in range(BPW // L):
            sl = pl.ds(m * L, L)
            rloc = lax.iota(jnp.int32, L) + m * L
            r_lo, r_hi = _pair(cols, rloc + par * REG, word_i[0, sl])
            acc = out_v[sl]
            acc = acc + r_lo * prod_v[d0, sl]
            acc = acc + r_hi * prod_v[d0 + 1, sl]
            out_v[sl] = acc

    _stage_and_fire(tables, 0, 0, sem0, base_i, gidx, cols, 1)
    _stage_and_fire(tables, 1, 1, sem1, base_i, gidx, cols, 1)

    def pipe_body(k, _):
        g_e = k * 2
        _drain(x_r, 0, sem0, cols, 1)
        accumulate(g_e, 0)

        @pl.when(k < NG // 2 - 1)
        def _fe():
            _stage_and_fire(tables, g_e + 2, 0, sem0, base_i, gidx, cols, 1)

        _drain(x_r, 1, sem1, cols, 1)
        accumulate(g_e + 1, 1)

        @pl.when(k < NG // 2 - 1)
        def _fo():
            _stage_and_fire(tables, g_e + 3, 1, sem1, base_i, gidx, cols, 1)

        return 0

    lax.fori_loop(0, NG // 2, pipe_body, 0)
    pltpu.sync_copy(out_v, out_hbm.at[pl.ds(wid * BPW, BPW)])


def kernel(triples, entity_emb, rel_emb):
    idx = triples.astype(jnp.int32)
    subj = idx[:, 0].reshape(NW, BPW)
    obj = idx[:, 1].reshape(NW, BPW)
    rel = idx[:, 2].reshape(NW, BPW)
    ent_x = _granule_view(entity_emb)
    rel_x = _granule_view(rel_emb)

    mesh = plsc.VectorSubcoreMesh(core_axis_name="c", subcore_axis_name="s")
    cp = pltpu.CompilerParams(
        needs_layout_passes=False, use_tc_tiling_on_sc=False)

    k1 = functools.partial(
        pl.kernel,
        mesh=mesh,
        compiler_params=cp,
        out_type=jax.ShapeDtypeStruct((NW, D, BPW), jnp.float32),
        scratch_types=[
            pltpu.VMEM((2, BPW), jnp.int32),
            pltpu.VMEM((2, BPW), jnp.int32),
            pltpu.VMEM((2, BPW), jnp.int32),
            pltpu.VMEM((2 * 2 * NCHUNK, CHUNK), jnp.int32),
            pltpu.VMEM((2 * 2 * REG, 16), jnp.int32),
            pltpu.VMEM((2, 2, BPW), jnp.float32),
            pltpu.SemaphoreType.DMA,
            pltpu.SemaphoreType.DMA,
        ],
    )(_k1_body)
    p = k1(subj, obj, ent_x)

    k2 = functools.partial(
        pl.kernel,
        mesh=mesh,
        compiler_params=cp,
        out_type=jax.ShapeDtypeStruct((B,), jnp.float32),
        scratch_types=[
            pltpu.VMEM((1, BPW), jnp.int32),
            pltpu.VMEM((1, BPW), jnp.int32),
            pltpu.VMEM((1, BPW), jnp.int32),
            pltpu.VMEM((2 * 1 * NCHUNK, CHUNK), jnp.int32),
            pltpu.VMEM((2 * 1 * REG, 16), jnp.int32),
            pltpu.VMEM((D, BPW), jnp.float32),
            pltpu.VMEM((BPW,), jnp.float32),
            pltpu.SemaphoreType.DMA,
            pltpu.SemaphoreType.DMA,
        ],
    )(_k2_body)
    scores = k2(rel, rel_x, p)
    return scores.reshape(B, 1)


# speedup vs baseline: 16.6846x; 16.6846x over previous
import functools

import jax
import jax.numpy as jnp
from jax import lax
from jax.experimental import pallas as pl
from jax.experimental.pallas import tpu as pltpu
from jax.experimental.pallas import tpu_sc as plsc

B = 16384
D = 32
L = 16
NC = 2
NS = 16
NW = NC * NS
BPW = B // NW
NPAD = 1000064
NTC = NPAD // 128
ROWS = 4 * NTC * 8 * 8
DG = 2
NG = D // DG
CHUNK = 128
NCHUNK = BPW // CHUNK
REG = DG * BPW


def _granule_view(table):
    tp = jnp.concatenate([table, jnp.zeros((NPAD - table.shape[0], D),
                                           table.dtype)], axis=0)
    et = tp.T
    return (et.reshape(4, 8, NTC, 128).transpose(0, 2, 1, 3)
            .reshape(ROWS, 16))


def _prep_indices(raw_i, base_i, lane_i, nt):
    for t in range(nt):
        for m in range(BPW // L):
            sl = pl.ds(m * L, L)
            n = raw_i[t, sl]
            base_i[t, sl] = (
                lax.shift_right_logical(n, 7) * 64
                + jnp.bitwise_and(lax.shift_right_logical(n, 4), 7))
            lane_i[t, sl] = jnp.bitwise_and(n, 15)


def _stage_and_fire(tables, g, par, sem, base_i, gidx, cols, nt):
    for dloc in range(DG):
        d = g * DG + dloc
        offs = (lax.shift_right_logical(d, 3) * (NTC * 64)
                + jnp.bitwise_and(d, 7) * 8)
        for t in range(nt):
            for j in range(NCHUNK):
                row = (par * nt + t) * DG * NCHUNK + dloc * NCHUNK + j
                for m in range(CHUNK // L):
                    sl = pl.ds(j * CHUNK + m * L, L)
                    gidx[row, pl.ds(m * L, L)] = base_i[t, sl] + offs
    for dloc in range(DG):
        for t in range(nt):
            for j in range(NCHUNK):
                row = (par * nt + t) * DG * NCHUNK + dloc * NCHUNK + j
                dst = pl.ds((par * nt + t) * REG + dloc * BPW + j * CHUNK,
                            CHUNK)
                pltpu.async_copy(tables[t].at[gidx.at[row]], cols.at[dst], sem)


def _drain(x_any, par, sem, cols, nt):
    for t in range(nt):
        reg = pl.ds((par * nt + t) * REG, REG)
        pltpu.make_async_copy(x_any.at[pl.ds(0, REG)], cols.at[reg],
                              sem).wait()


def _k1_body(subj_hbm, obj_hbm, x_e, p_hbm,
             raw_i, base_i, lane_i, gidx, cols, prod_v, sem0, sem1):
    wid = lax.axis_index("s") * NC + lax.axis_index("c")
    pltpu.sync_copy(subj_hbm.at[wid], raw_i.at[0])
    pltpu.sync_copy(obj_hbm.at[wid], raw_i.at[1])
    _prep_indices(raw_i, base_i, lane_i, 2)
    tables = (x_e, x_e)

    def accumulate(g, par):
        for dloc in range(DG):
            for m in range(BPW // L):
                sl = pl.ds(m * L, L)
                rloc = lax.iota(jnp.int32, L) + m * L
                vals = []
                for t in range(2):
                    ridx = rloc + ((par * 2 + t) * REG + dloc * BPW)
                    vals.append(plsc.load_gather(cols, [ridx, lane_i[t, sl]]))
                prod_v[par, dloc, sl] = vals[0] * vals[1]
        pltpu.sync_copy(prod_v.at[par], p_hbm.at[wid, pl.ds(g * DG, DG)])

    _stage_and_fire(tables, 0, 0, sem0, base_i, gidx, cols, 2)
    _stage_and_fire(tables, 1, 1, sem1, base_i, gidx, cols, 2)

    def pipe_body(k, _):
        g_e = k * 2
        _drain(x_e, 0, sem0, cols, 2)
        accumulate(g_e, 0)

        @pl.when(k < NG // 2 - 1)
        def _fe():
            _stage_and_fire(tables, g_e + 2, 0, sem0, base_i, gidx, cols, 2)

        _drain(x_e, 1, sem1, cols, 2)
        accumulate(g_e + 1, 1)

        @pl.when(k < NG // 2 - 1)
        def _fo():
            _stage_and_fire(tables, g_e + 3, 1, sem1, base_i, gidx, cols, 2)

        return 0

    lax.fori_loop(0, NG // 2, pipe_body, 0)


def _k2_body(rel_hbm, x_r, p_hbm, out_hbm,
             raw_i, base_i, lane_i, gidx, cols, prod_v, out_v, sem0, sem1):
    wid = lax.axis_index("s") * NC + lax.axis_index("c")
    pltpu.sync_copy(rel_hbm.at[wid], raw_i.at[0])
    pltpu.sync_copy(p_hbm.at[wid], prod_v)
    _prep_indices(raw_i, base_i, lane_i, 1)
    tables = (x_r,)

    zero = jnp.zeros((L,), jnp.float32)
    for m in range(BPW // L):
        out_v[pl.ds(m * L, L)] = zero

    def accumulate(g, par):
        for m in range(BPW // L):
            sl = pl.ds(m * L, L)
            acc = out_v[sl]
            rloc = lax.iota(jnp.int32, L) + m * L
            for dloc in range(DG):
                d = g * DG + dloc
                ridx = rloc + (par * REG + dloc * BPW)
                rv = plsc.load_gather(cols, [ridx, lane_i[0, sl]])
                acc = acc + rv * prod_v[d, sl]
            out_v[sl] = acc

    _stage_and_fire(tables, 0, 0, sem0, base_i, gidx, cols, 1)
    _stage_and_fire(tables, 1, 1, sem1, base_i, gidx, cols, 1)

    def pipe_body(k, _):
        g_e = k * 2
        _drain(x_r, 0, sem0, cols, 1)
        accumulate(g_e, 0)

        @pl.when(k < NG // 2 - 1)
        def _fe():
            _stage_and_fire(tables, g_e + 2, 0, sem0, base_i, gidx, cols, 1)

        _drain(x_r, 1, sem1, cols, 1)
        accumulate(g_e + 1, 1)

        @pl.when(k < NG // 2 - 1)
        def _fo():
            _stage_and_fire(tables, g_e + 3, 1, sem1, base_i, gidx, cols, 1)

        return 0

    lax.fori_loop(0, NG // 2, pipe_body, 0)
    pltpu.sync_copy(out_v, out_hbm.at[pl.ds(wid * BPW, BPW)])


def kernel(triples, entity_emb, rel_emb):
    idx = triples.astype(jnp.int32)
    subj = idx[:, 0].reshape(NW, BPW)
    obj = idx[:, 1].reshape(NW, BPW)
    rel = idx[:, 2].reshape(NW, BPW)
    ent_x = _granule_view(entity_emb)
    rel_x = _granule_view(rel_emb)

    mesh = plsc.VectorSubcoreMesh(core_axis_name="c", subcore_axis_name="s")
    cp = pltpu.CompilerParams(
        needs_layout_passes=False, use_tc_tiling_on_sc=False)

    k1 = functools.partial(
        pl.kernel,
        mesh=mesh,
        compiler_params=cp,
        out_type=jax.ShapeDtypeStruct((NW, D, BPW), jnp.float32),
        scratch_types=[
            pltpu.VMEM((2, BPW), jnp.int32),
            pltpu.VMEM((2, BPW), jnp.int32),
            pltpu.VMEM((2, BPW), jnp.int32),
            pltpu.VMEM((2 * 2 * DG * NCHUNK, CHUNK), jnp.int32),
            pltpu.VMEM((2 * 2 * REG, 16), jnp.float32),
            pltpu.VMEM((2, DG, BPW), jnp.float32),
            pltpu.SemaphoreType.DMA,
            pltpu.SemaphoreType.DMA,
        ],
    )(_k1_body)
    p = k1(subj, obj, ent_x)

    k2 = functools.partial(
        pl.kernel,
        mesh=mesh,
        compiler_params=cp,
        out_type=jax.ShapeDtypeStruct((B,), jnp.float32),
        scratch_types=[
            pltpu.VMEM((1, BPW), jnp.int32),
            pltpu.VMEM((1, BPW), jnp.int32),
            pltpu.VMEM((1, BPW), jnp.int32),
            pltpu.VMEM((2 * 1 * DG * NCHUNK, CHUNK), jnp.int32),
            pltpu.VMEM((2 * 1 * REG, 16), jnp.float32),
            pltpu.VMEM((D, BPW), jnp.float32),
            pltpu.VMEM((BPW,), jnp.float32),
            pltpu.SemaphoreType.DMA,
            pltpu.SemaphoreType.DMA,
        ],
    )(_k2_body)
    scores = k2(rel, rel_x, p)
    return scores.reshape(B, 1)
